# Initial kernel scaffold; baseline (speedup 1.0000x reference)
import functools

import jax
import jax.numpy as jnp
from jax import lax
from jax.experimental import pallas as pl
from jax.experimental.pallas import tpu as pltpu
from jax.experimental.pallas import tpu_sc as plsc

VOCAB = 100000
DIM = 128
BATCH = 16384
NEG = 20
CLAMP = 4.0

NCORE = 2
NSUB = 16
NWORK = NCORE * NSUB
EPW = BATCH // NWORK
CB = 16
NCHUNK = EPW // CB
ROWS_PER_E = NEG + 1
CROWS = CB * ROWS_PER_E
SCORE_W = 32

_mesh = plsc.VectorSubcoreMesh(core_axis_name="c", subcore_axis_name="s")


@functools.partial(
    pl.kernel,
    mesh=_mesh,
    out_type=jax.ShapeDtypeStruct((BATCH, SCORE_W), jnp.float32),
    scratch_types=[
        pltpu.VMEM((EPW,), jnp.int32),
        pltpu.VMEM((EPW * ROWS_PER_E,), jnp.int32),
        pltpu.VMEM((CB, DIM), jnp.float32),
        pltpu.VMEM((CROWS, DIM), jnp.float32),
        pltpu.VMEM((CB, SCORE_W), jnp.float32),
        pltpu.SemaphoreType.DMA,
    ],
)
def _sc_scores(in_idx_hbm, out_idx_hbm, emb_in_hbm, emb_out_hbm, scores_hbm,
               inidx_w, outidx_w, inrows, outrows, score, sem):
    cid = lax.axis_index("c")
    sid = lax.axis_index("s")
    wid = sid * NCORE + cid
    ebase = wid * EPW
    pltpu.sync_copy(in_idx_hbm.at[pl.ds(ebase, EPW)], inidx_w)
    pltpu.sync_copy(out_idx_hbm.at[pl.ds(ebase * ROWS_PER_E, EPW * ROWS_PER_E)],
                    outidx_w)
    lane = lax.broadcasted_iota(jnp.int32, (16,), 0)

    def chunk(c, carry):
        cb = c * CB
        cp_in = pltpu.async_copy(
            emb_in_hbm.at[inidx_w.at[pl.ds(cb, CB)]], inrows, sem)
        cps = []
        for g, ln in ((0, 128), (128, 128), (256, 80)):
            cps.append(pltpu.async_copy(
                emb_out_hbm.at[outidx_w.at[pl.ds(c * CROWS + g, ln)]],
                outrows.at[pl.ds(g, ln)], sem))
        cp_in.wait()
        for cp in cps:
            cp.wait()

        def elem(e, carry2):
            vin = [inrows[e, pl.ds(k * 16, 16)] for k in range(8)]
            srow0 = jnp.zeros((16,), jnp.float32)
            srow1 = jnp.zeros((16,), jnp.float32)
            for j in range(ROWS_PER_E):
                r = e * ROWS_PER_E + j
                acc = vin[0] * outrows[r, pl.ds(0, 16)]
                for k in range(1, 8):
                    acc = acc + vin[k] * outrows[r, pl.ds(k * 16, 16)]
                t = jnp.clip(jnp.sum(acc), -CLAMP, CLAMP)
                if j == 0:
                    t = -t
                if j < 16:
                    srow0 = jnp.where(lane == j, t, srow0)
                else:
                    srow1 = jnp.where(lane == (j - 16), t, srow1)
            score[e, pl.ds(0, 16)] = srow0
            score[e, pl.ds(16, 16)] = srow1
            return carry2

        lax.fori_loop(0, CB, elem, 0)
        pltpu.sync_copy(score, scores_hbm.at[pl.ds(ebase + cb, CB)])
        return carry

    lax.fori_loop(0, NCHUNK, chunk, 0)


def _finish_body(s_ref, o_ref):
    x = s_ref[...]
    col = lax.broadcasted_iota(jnp.int32, x.shape, 1)
    t = jnp.where(col < ROWS_PER_E, x, -1e30)
    sp = jnp.maximum(t, 0.0) + jnp.log1p(jnp.exp(-jnp.abs(t)))
    o_ref[...] = jnp.sum(sp, axis=1)


_finish = pl.pallas_call(
    _finish_body,
    grid=(BATCH // 1024,),
    in_specs=[pl.BlockSpec((1024, SCORE_W), lambda i: (i, 0))],
    out_specs=pl.BlockSpec((1024,), lambda i: (i,)),
    out_shape=jax.ShapeDtypeStruct((BATCH,), jnp.float32),
)


def kernel(inputs, positiveOutputs, negativeOutputs, emb_in, emb_out):
    inputs = inputs.astype(jnp.int32)
    out_idx = jnp.concatenate(
        [positiveOutputs.astype(jnp.int32)[:, None],
         negativeOutputs.astype(jnp.int32)], axis=1).reshape(-1)
    scores = _sc_scores(inputs, out_idx, emb_in, emb_out)
    return _finish(scores)


# speedup vs baseline: 7.2875x; 7.2875x over previous
import functools

import jax
import jax.numpy as jnp
from jax import lax
from jax.experimental import pallas as pl
from jax.experimental.pallas import tpu as pltpu
from jax.experimental.pallas import tpu_sc as plsc

VOCAB = 100000
DIM = 128
BATCH = 16384
NEG = 20
CLAMP = 4.0

NCORE = 2
NSUB = 16
NWORK = NCORE * NSUB
EPW = BATCH // NWORK
CB = 16
NCHUNK = EPW // CB
ROWS_PER_E = NEG + 1
CROWS = CB * ROWS_PER_E
SCORE_W = 32

_mesh = plsc.VectorSubcoreMesh(core_axis_name="c", subcore_axis_name="s")

_GDN = lax.GatherDimensionNumbers(
    offset_dims=(), collapsed_slice_dims=(0,), start_index_map=(0,))


def _shuffle(x, idx):
    return lax.gather(x, idx[:, None], dimension_numbers=_GDN,
                      slice_sizes=(1,),
                      mode=lax.GatherScatterMode.PROMISE_IN_BOUNDS)


@functools.partial(
    pl.kernel,
    mesh=_mesh,
    out_type=jax.ShapeDtypeStruct((BATCH, SCORE_W), jnp.float32),
    scratch_types=[
        pltpu.VMEM((EPW,), jnp.int32),
        pltpu.VMEM((EPW * ROWS_PER_E,), jnp.int32),
        pltpu.VMEM((CB, DIM), jnp.float32),
        pltpu.VMEM((CROWS, DIM), jnp.float32),
        pltpu.VMEM((CB, SCORE_W), jnp.float32),
        pltpu.SemaphoreType.DMA,
    ],
)
def _sc_scores(in_idx_hbm, out_idx_hbm, emb_in_hbm, emb_out_hbm, scores_hbm,
               inidx_w, outidx_w, inrows, outrows, score, sem):
    cid = lax.axis_index("c")
    sid = lax.axis_index("s")
    wid = sid * NCORE + cid
    ebase = wid * EPW
    pltpu.sync_copy(in_idx_hbm.at[pl.ds(ebase, EPW)], inidx_w)
    pltpu.sync_copy(out_idx_hbm.at[pl.ds(ebase * ROWS_PER_E, EPW * ROWS_PER_E)],
                    outidx_w)
    lane = lax.broadcasted_iota(jnp.int32, (16,), 0)
    perms = [lane ^ 8, lane ^ 4, lane ^ 2, lane ^ 1]

    def chunk(c, carry):
        cb = c * CB
        cp_in = pltpu.async_copy(
            emb_in_hbm.at[inidx_w.at[pl.ds(cb, CB)]], inrows, sem)
        cps = []
        for g, ln in ((0, 128), (128, 128), (256, 80)):
            cps.append(pltpu.async_copy(
                emb_out_hbm.at[outidx_w.at[pl.ds(c * CROWS + g, ln)]],
                outrows.at[pl.ds(g, ln)], sem))
        cp_in.wait()
        for cp in cps:
            cp.wait()

        def elem(e, carry2):
            vin = [inrows[e, pl.ds(k * 16, 16)] for k in range(8)]
            srow0 = jnp.zeros((16,), jnp.float32)
            srow1 = jnp.zeros((16,), jnp.float32)
            for j in range(ROWS_PER_E):
                r = e * ROWS_PER_E + j
                acc = vin[0] * outrows[r, pl.ds(0, 16)]
                for k in range(1, 8):
                    acc = acc + vin[k] * outrows[r, pl.ds(k * 16, 16)]
                for p in perms:
                    acc = acc + _shuffle(acc, p)
                t = jnp.clip(acc, -CLAMP, CLAMP)
                if j == 0:
                    t = -t
                if j < 16:
                    srow0 = jnp.where(lane == j, t, srow0)
                else:
                    srow1 = jnp.where(lane == (j - 16), t, srow1)
            score[e, pl.ds(0, 16)] = srow0
            score[e, pl.ds(16, 16)] = srow1
            return carry2

        lax.fori_loop(0, CB, elem, 0)
        pltpu.sync_copy(score, scores_hbm.at[pl.ds(ebase + cb, CB)])
        return carry

    lax.fori_loop(0, NCHUNK, chunk, 0)


def _finish_body(s_ref, o_ref):
    x = s_ref[...]
    col = lax.broadcasted_iota(jnp.int32, x.shape, 1)
    t = jnp.where(col < ROWS_PER_E, x, -1e30)
    sp = jnp.maximum(t, 0.0) + jnp.log1p(jnp.exp(-jnp.abs(t)))
    o_ref[...] = jnp.sum(sp, axis=1)


_finish = pl.pallas_call(
    _finish_body,
    grid=(BATCH // 1024,),
    in_specs=[pl.BlockSpec((1024, SCORE_W), lambda i: (i, 0))],
    out_specs=pl.BlockSpec((1024,), lambda i: (i,)),
    out_shape=jax.ShapeDtypeStruct((BATCH,), jnp.float32),
)


def kernel(inputs, positiveOutputs, negativeOutputs, emb_in, emb_out):
    inputs = inputs.astype(jnp.int32)
    out_idx = jnp.concatenate(
        [positiveOutputs.astype(jnp.int32)[:, None],
         negativeOutputs.astype(jnp.int32)], axis=1).reshape(-1)
    scores = _sc_scores(inputs, out_idx, emb_in, emb_out)
    return _finish(scores)


# speedup vs baseline: 11.0522x; 1.5166x over previous
import functools

import jax
import jax.numpy as jnp
from jax import lax
from jax.experimental import pallas as pl
from jax.experimental.pallas import tpu as pltpu
from jax.experimental.pallas import tpu_sc as plsc

VOCAB = 100000
DIM = 128
BATCH = 16384
NEG = 20
CLAMP = 4.0

NCORE = 2
NSUB = 16
NWORK = NCORE * NSUB
EPW = BATCH // NWORK
CB = 16
NCHUNK = EPW // CB
ROWS_PER_E = NEG + 1
CROWS = CB * ROWS_PER_E
SCORE_W = 32

_mesh = plsc.VectorSubcoreMesh(core_axis_name="c", subcore_axis_name="s")

_GDN = lax.GatherDimensionNumbers(
    offset_dims=(), collapsed_slice_dims=(0,), start_index_map=(0,))


def _shuffle(x, idx):
    return lax.gather(x, idx[:, None], dimension_numbers=_GDN,
                      slice_sizes=(1,),
                      mode=lax.GatherScatterMode.PROMISE_IN_BOUNDS)


@functools.partial(
    pl.kernel,
    mesh=_mesh,
    out_type=jax.ShapeDtypeStruct((BATCH, SCORE_W), jnp.float32),
    scratch_types=[
        pltpu.VMEM((EPW,), jnp.int32),
        pltpu.VMEM((EPW * ROWS_PER_E,), jnp.int32),
        pltpu.VMEM((2, CB, DIM), jnp.float32),
        pltpu.VMEM((2, CROWS, DIM), jnp.float32),
        pltpu.VMEM((CB, SCORE_W), jnp.float32),
        pltpu.SemaphoreType.DMA,
        pltpu.SemaphoreType.DMA,
    ],
)
def _sc_scores(in_idx_hbm, out_idx_hbm, emb_in_hbm, emb_out_hbm, scores_hbm,
               inidx_w, outidx_w, inrows2, outrows2, score, semA, semB):
    cid = lax.axis_index("c")
    sid = lax.axis_index("s")
    wid = sid * NCORE + cid
    ebase = wid * EPW
    pltpu.sync_copy(in_idx_hbm.at[pl.ds(ebase, EPW)], inidx_w)
    pltpu.sync_copy(out_idx_hbm.at[pl.ds(ebase * ROWS_PER_E, EPW * ROWS_PER_E)],
                    outidx_w)
    lane = lax.broadcasted_iota(jnp.int32, (16,), 0)
    perms = [lane ^ 8, lane ^ 4, lane ^ 2, lane ^ 1]

    bufs = (
        (inrows2.at[0], outrows2.at[0], semA),
        (inrows2.at[1], outrows2.at[1], semB),
    )

    def copies(c, buf):
        inb, outb, sem = buf
        cps = [pltpu.make_async_copy(
            emb_in_hbm.at[inidx_w.at[pl.ds(c * CB, CB)]], inb, sem)]
        for g, ln in ((0, 128), (128, 128), (256, 80)):
            cps.append(pltpu.make_async_copy(
                emb_out_hbm.at[outidx_w.at[pl.ds(c * CROWS + g, ln)]],
                outb.at[pl.ds(g, ln)], sem))
        return cps

    def issue(c, buf):
        for cp in copies(c, buf):
            cp.start()

    def drain(c, buf):
        for cp in copies(c, buf):
            cp.wait()

    def compute(c, buf):
        inb, outb, _ = buf

        def elem(e, carry2):
            vin = [inb[e, pl.ds(k * 16, 16)] for k in range(8)]
            srow0 = jnp.zeros((16,), jnp.float32)
            srow1 = jnp.zeros((16,), jnp.float32)
            for j in range(ROWS_PER_E):
                r = e * ROWS_PER_E + j
                acc = vin[0] * outb[r, pl.ds(0, 16)]
                for k in range(1, 8):
                    acc = acc + vin[k] * outb[r, pl.ds(k * 16, 16)]
                for p in perms:
                    acc = acc + _shuffle(acc, p)
                t = jnp.clip(acc, -CLAMP, CLAMP)
                if j == 0:
                    t = -t
                if j < 16:
                    srow0 = jnp.where(lane == j, t, srow0)
                else:
                    srow1 = jnp.where(lane == (j - 16), t, srow1)
            score[e, pl.ds(0, 16)] = srow0
            score[e, pl.ds(16, 16)] = srow1
            return carry2

        lax.fori_loop(0, CB, elem, 0)
        pltpu.sync_copy(score, scores_hbm.at[pl.ds(ebase + c * CB, CB)])

    issue(0, bufs[0])

    def body2(i, carry):
        c0 = 2 * i
        issue(c0 + 1, bufs[1])
        drain(c0, bufs[0])
        compute(c0, bufs[0])
        pl.when(i < NCHUNK // 2 - 1)(lambda: issue(c0 + 2, bufs[0]))
        drain(c0 + 1, bufs[1])
        compute(c0 + 1, bufs[1])
        return carry

    lax.fori_loop(0, NCHUNK // 2, body2, 0)


def _finish_body(s_ref, o_ref):
    x = s_ref[...]
    col = lax.broadcasted_iota(jnp.int32, x.shape, 1)
    t = jnp.where(col < ROWS_PER_E, x, -1e30)
    sp = jnp.maximum(t, 0.0) + jnp.log1p(jnp.exp(-jnp.abs(t)))
    o_ref[...] = jnp.sum(sp, axis=1)


_finish = pl.pallas_call(
    _finish_body,
    grid=(BATCH // 1024,),
    in_specs=[pl.BlockSpec((1024, SCORE_W), lambda i: (i, 0))],
    out_specs=pl.BlockSpec((1024,), lambda i: (i,)),
    out_shape=jax.ShapeDtypeStruct((BATCH,), jnp.float32),
)


def kernel(inputs, positiveOutputs, negativeOutputs, emb_in, emb_out):
    inputs = inputs.astype(jnp.int32)
    out_idx = jnp.concatenate(
        [positiveOutputs.astype(jnp.int32)[:, None],
         negativeOutputs.astype(jnp.int32)], axis=1).reshape(-1)
    scores = _sc_scores(inputs, out_idx, emb_in, emb_out)
    return _finish(scores)
